# Initial kernel scaffold; baseline (speedup 1.0000x reference)
import jax
import jax.numpy as jnp
from jax.experimental import pallas as pl


def kernel(patch_embeddings, coordinates, edge_index, edge_attr, tp_W1, tp_b1, tp_g1, tp_be1, tp_W2, tp_b2, tp_g2, tp_be2, g1_Wq, g1_bq, g1_Wk, g1_bk, g1_Wv, g1_bv, g1_We, g1_Ws, g1_bs, g1_ng, g1_nb, g2_Wq, g2_bq, g2_Wk, g2_bk, g2_Wv, g2_bv, g2_We, g2_Ws, g2_bs, g2_ng, g2_nb, pr_W1, pr_b1, pr_g, pr_be, pr_W2, pr_b2):
    raise NotImplementedError("write your pallas kernel here")


# speedup vs baseline: 18.2530x; 18.2530x over previous
import functools
import math

import jax
import jax.numpy as jnp
import numpy as np
from jax import lax
from jax.experimental import pallas as pl
from jax.experimental.pallas import tpu as pltpu
from jax.experimental.pallas import tpu_sc as plsc

_N = 10000
_E = 640000
_HEADS = 4
_HD = 24
_F = 96
_EDIM = 4

_ROWS = 1000
_GRID = _N // _ROWS

_NW = 32
_EPW = _E // _NW
_C = 16
_NCHUNK = _EPW // _C
_EPB = 2000
_CPB = _EPB // _C
_ACCW = 128
_ZCH = 16
_NZCH = _N // _ZCH


def _ln(x, g, b):
    m = jnp.mean(x, axis=-1, keepdims=True)
    v = jnp.mean((x - m) ** 2, axis=-1, keepdims=True)
    return g * (x - m) / jnp.sqrt(v + 1e-5) + b


_HEADCOL = np.repeat(np.eye(4, dtype=np.float32), 24, axis=0)
_SDEN = np.zeros((_ACCW, 96), np.float32)
_SDEN[96:100, :] = _HEADCOL.T
_SACC = np.zeros((_ACCW, 96), np.float32)
_SACC[:96, :] = np.eye(96, dtype=np.float32)


def _mk_M(We):
    return (jnp.transpose(We).reshape(96, 1, 4) * _HEADCOL.reshape(96, 4, 1)).reshape(96, 16)


def _embed_body(pe, co, W1, b1, g1, be1, W2, b2, g2, be2, h_o, pe_o):
    x = pe[...]
    t = jnp.maximum(jnp.dot(x, W1[...]) + b1[...], 0.0)
    t = _ln(t, g1[...], be1[...])
    t = jnp.maximum(jnp.dot(t, W2[...]) + b2[...], 0.0)
    h_o[...] = _ln(t, g2[...], be2[...])
    c = co[...]
    lane = lax.broadcasted_iota(jnp.int32, (1, 32), 1)
    i16 = lane % 16
    freq = jnp.exp((i16 // 2).astype(jnp.float32) * (2.0 * -math.log(10000.0) / 16.0))
    phase = (i16 % 2).astype(jnp.float32) * (math.pi / 2.0)
    xsel = (lane < 16).astype(jnp.float32)
    c_sel = c[:, 0:1] * xsel + c[:, 1:2] * (1.0 - xsel)
    pe_o[...] = jnp.sin(c_sel * freq + phase)


def _embed(pe, co, W1, b1, g1, be1, W2, b2, g2, be2):
    row = lambda i: (i, 0)
    full = lambda i: (0, 0)
    return pl.pallas_call(
        _embed_body,
        grid=(_GRID,),
        in_specs=[
            pl.BlockSpec((_ROWS, 768), row),
            pl.BlockSpec((_ROWS, 2), row),
            pl.BlockSpec((768, 2048), full),
            pl.BlockSpec((1, 2048), full),
            pl.BlockSpec((1, 2048), full),
            pl.BlockSpec((1, 2048), full),
            pl.BlockSpec((2048, 64), full),
            pl.BlockSpec((1, 64), full),
            pl.BlockSpec((1, 64), full),
            pl.BlockSpec((1, 64), full),
        ],
        out_specs=[pl.BlockSpec((_ROWS, 64), row), pl.BlockSpec((_ROWS, 32), row)],
        out_shape=[
            jax.ShapeDtypeStruct((_N, 64), jnp.float32),
            jax.ShapeDtypeStruct((_N, 32), jnp.float32),
        ],
    )(pe, co, W1, b1, g1, be1, W2, b2, g2, be2)


def _qkv_body(x, Wq, bq, Wk, bk, Wv, bv, M, d_o, s_o):
    xv = x[...]
    q = jnp.dot(xv, Wq[...]) + bq[...]
    qe = jnp.dot(q, M[...])
    k = jnp.dot(xv, Wk[...]) + bk[...]
    v = jnp.dot(xv, Wv[...]) + bv[...]
    zp = jnp.zeros((xv.shape[0], 16), jnp.float32)
    d_o[...] = jnp.concatenate([q, qe, zp], axis=1)
    s_o[...] = jnp.concatenate([k, v, zp, zp, zp, zp], axis=1)


def _qkv_pack(x, Wq, bq, Wk, bk, Wv, bv, M):
    row = lambda i: (i, 0)
    full = lambda i: (0, 0)
    return pl.pallas_call(
        _qkv_body,
        grid=(_GRID,),
        in_specs=[
            pl.BlockSpec((_ROWS, _F), row),
            pl.BlockSpec((_F, _F), full),
            pl.BlockSpec((1, _F), full),
            pl.BlockSpec((_F, _F), full),
            pl.BlockSpec((1, _F), full),
            pl.BlockSpec((_F, _F), full),
            pl.BlockSpec((1, _F), full),
            pl.BlockSpec((_F, 16), full),
        ],
        out_specs=[
            pl.BlockSpec((_ROWS, 128), row),
            pl.BlockSpec((_ROWS, 256), row),
        ],
        out_shape=[
            jax.ShapeDtypeStruct((_N, 128), jnp.float32),
            jax.ShapeDtypeStruct((_N, 256), jnp.float32),
        ],
    )(x, Wq, bq, Wk, bk, Wv, bv, M)


def _edge_body(dtab, stab, eaf, src, dst, out,
               sidx, didx, eab, drow, srow, scaled, acc,
               semd0, semd1, sems0, sems1, sema0, sema1):
    cid = lax.axis_index("c")
    sid = lax.axis_index("s")
    wid = sid * 2 + cid

    z16 = jnp.zeros((16,), jnp.float32)
    lanes = lax.iota(jnp.int32, 16)
    inv = jnp.float32(1.0 / math.sqrt(_HD))

    def zrow(r, _):
        for j in range(_ACCW // 16):
            scaled[r, pl.ds(j * 16, 16)] = z16
        return 0

    lax.fori_loop(0, 2 * _C, zrow, 0)

    def zchunk(j, _):
        idx = sid + 16 * j

        @pl.when(idx < _NZCH)
        def _():
            off = pl.multiple_of(idx * _ZCH, 8)
            pltpu.sync_copy(scaled.at[pl.ds(0, _ZCH)], acc.at[pl.ds(off, _ZCH)])

        return 0

    lax.fori_loop(0, (_NZCH + 15) // 16, zchunk, 0)
    plsc.subcore_barrier()

    zi16 = jnp.zeros((16,), jnp.int32)

    def step(j, carry):
        dprev, ea0, ea1, ea2, ea3 = carry
        par = j % 2
        parc = (j - 1) % 2
        blk = j // _CPB
        jbe = j - blk * _CPB

        @pl.when(jnp.logical_and(j < _NCHUNK, jbe == 0))
        def _():
            base = pl.multiple_of(wid * _EPW + blk * _EPB, 8)
            pltpu.sync_copy(src.at[pl.ds(base, _EPB)], sidx)
            pltpu.sync_copy(dst.at[pl.ds(base, _EPB)], didx)
            pltpu.sync_copy(eaf.at[pl.ds(base * _EDIM, _EPB * _EDIM)], eab)

        inb = j < _NCHUNK
        jsafe = jnp.where(inb, jbe, 0)
        cbase = jsafe * _C + lanes
        svec = plsc.load_gather(sidx, [cbase])
        dvec = plsc.load_gather(didx, [cbase])
        eidx = cbase * _EDIM
        eacn = [plsc.load_gather(eab, [eidx + d]) for d in range(_EDIM)]

        @pl.when(jnp.logical_and(j >= 1, parc == 0))
        def _():
            pltpu.make_async_copy(dtab.at[dprev], drow.at[pl.ds(0, _C)],
                                  semd0).wait()
            pltpu.make_async_copy(stab.at[dprev], srow.at[pl.ds(0, _C)],
                                  sems0).wait()

        @pl.when(jnp.logical_and(j >= 1, parc == 1))
        def _():
            pltpu.make_async_copy(dtab.at[dprev], drow.at[pl.ds(_C, _C)],
                                  semd1).wait()
            pltpu.make_async_copy(stab.at[dprev], srow.at[pl.ds(_C, _C)],
                                  sems1).wait()

        @pl.when(jnp.logical_and(inb, par == 0))
        def _():
            pltpu.async_copy(dtab.at[dvec], drow.at[pl.ds(0, _C)], semd0)
            pltpu.async_copy(stab.at[svec], srow.at[pl.ds(0, _C)], sems0)

        @pl.when(jnp.logical_and(inb, par == 1))
        def _():
            pltpu.async_copy(dtab.at[dvec], drow.at[pl.ds(_C, _C)], semd1)
            pltpu.async_copy(stab.at[svec], srow.at[pl.ds(_C, _C)], sems1)

        @pl.when(j >= 1)
        def _():
            rl = lanes + parc * _C

            @pl.when(jnp.logical_and(j >= 3, parc == 0))
            def _():
                pltpu.make_async_copy(scaled.at[pl.ds(0, _C)],
                                      acc.at[dprev], sema0).wait()

            @pl.when(jnp.logical_and(j >= 3, parc == 1))
            def _():
                pltpu.make_async_copy(scaled.at[pl.ds(_C, _C)],
                                      acc.at[dprev], sema1).wait()

            a = [z16, z16, z16, z16]
            for c in range(_F):
                colv = jnp.full((16,), c, jnp.int32)
                qc = plsc.load_gather(drow, [rl, colv])
                kc = plsc.load_gather(srow, [rl, colv])
                a[c // _HD] = a[c // _HD] + qc * kc
            eac = [ea0, ea1, ea2, ea3]
            ex = []
            for h in range(_HEADS):
                ah = a[h]
                for d in range(_EDIM):
                    qe = plsc.load_gather(
                        drow, [rl, jnp.full((16,), _F + h * 4 + d, jnp.int32)])
                    ah = ah + eac[d] * qe
                ex.append(jnp.exp(ah * inv))
            for c in range(_F):
                vc = plsc.load_gather(
                    srow, [rl, jnp.full((16,), _F + c, jnp.int32)])
                plsc.store_scatter(scaled, [rl, jnp.full((16,), c, jnp.int32)],
                                   vc * ex[c // _HD])
            for h in range(_HEADS):
                plsc.store_scatter(scaled,
                                   [rl, jnp.full((16,), _F + h, jnp.int32)],
                                   ex[h])
                for d in range(_EDIM):
                    plsc.store_scatter(
                        scaled,
                        [rl, jnp.full((16,), 100 + h * 4 + d, jnp.int32)],
                        ex[h] * eac[d])
            @pl.when(parc == 0)
            def _():
                pltpu.async_copy(scaled.at[pl.ds(0, _C)], acc.at[dprev],
                                 sema0, add=True)

            @pl.when(parc == 1)
            def _():
                pltpu.async_copy(scaled.at[pl.ds(_C, _C)], acc.at[dprev],
                                 sema1, add=True)

        return (dvec, eacn[0], eacn[1], eacn[2], eacn[3])

    lax.fori_loop(0, _NCHUNK + 1, step,
                  (zi16, z16, z16, z16, z16))
    pltpu.make_async_copy(scaled.at[pl.ds(0, _C)], acc.at[lanes], sema0).wait()
    pltpu.make_async_copy(scaled.at[pl.ds(_C, _C)], acc.at[lanes], sema1).wait()
    plsc.subcore_barrier()

    def fchunk(j, _):
        idx = sid + 16 * j

        @pl.when(idx < _NZCH)
        def _():
            off = pl.multiple_of(idx * _ZCH, 8)
            pltpu.sync_copy(acc.at[pl.ds(off, _ZCH)],
                            out.at[cid, pl.ds(off, _ZCH)])

        return 0

    lax.fori_loop(0, (_NZCH + 15) // 16, fchunk, 0)


def _edge_pass(dtab, stab, ea, src, dst):
    mesh = plsc.VectorSubcoreMesh(core_axis_name="c", subcore_axis_name="s")
    kern = pl.kernel(
        _edge_body,
        out_type=jax.ShapeDtypeStruct((2, _N, _ACCW), jnp.float32),
        mesh=mesh,
        compiler_params=pltpu.CompilerParams(needs_layout_passes=False),
        scratch_types=[
            pltpu.VMEM((_EPB,), jnp.int32),
            pltpu.VMEM((_EPB,), jnp.int32),
            pltpu.VMEM((_EPB * _EDIM,), jnp.float32),
            pltpu.VMEM((2 * _C, 128), jnp.float32),
            pltpu.VMEM((2 * _C, 256), jnp.float32),
            pltpu.VMEM((2 * _C, _ACCW), jnp.float32),
            pltpu.VMEM_SHARED((_N, _ACCW), jnp.float32),
            pltpu.SemaphoreType.DMA,
            pltpu.SemaphoreType.DMA,
            pltpu.SemaphoreType.DMA,
            pltpu.SemaphoreType.DMA,
            pltpu.SemaphoreType.DMA,
            pltpu.SemaphoreType.DMA,
        ],
    )
    return kern(dtab, stab, ea.reshape(-1), src, dst)


def _fin_body(p0, p1, x, Ws, bs, ng, nb, SaccB, Sden, o):
    s = p0[...] + p1[...]
    num = jnp.dot(s, SaccB[...])
    den = jnp.dot(s, Sden[...])
    agg = num / (den + 1e-16)
    xv = x[...]
    y = agg + jnp.dot(xv, Ws[...]) + bs[...]
    y = jnp.maximum(y, 0.0)
    y = _ln(y, ng[...], nb[...])
    o[...] = xv + y


def _finalize(p0, p1, x, Ws, bs, ng, nb, SaccB, Sden):
    row = lambda i: (i, 0)
    full = lambda i: (0, 0)
    return pl.pallas_call(
        _fin_body,
        grid=(_GRID,),
        in_specs=[
            pl.BlockSpec((_ROWS, _ACCW), row),
            pl.BlockSpec((_ROWS, _ACCW), row),
            pl.BlockSpec((_ROWS, _F), row),
            pl.BlockSpec((_F, _F), full),
            pl.BlockSpec((1, _F), full),
            pl.BlockSpec((1, _F), full),
            pl.BlockSpec((1, _F), full),
            pl.BlockSpec((_ACCW, _F), full),
            pl.BlockSpec((_ACCW, _F), full),
        ],
        out_specs=pl.BlockSpec((_ROWS, _F), row),
        out_shape=jax.ShapeDtypeStruct((_N, _F), jnp.float32),
    )(p0, p1, x, Ws, bs, ng, nb, SaccB, Sden)


def _head_body(xin, W1, b1, g, be, W2, b2, o):
    y = jnp.maximum(jnp.dot(xin[...], W1[...]) + b1[...], 0.0)
    y = _ln(y, g[...], be[...])
    o[...] = jnp.dot(y, W2[...]) + b2[...]


def _head(xin, W1, b1, g, be, W2, b2):
    row = lambda i: (i, 0)
    full = lambda i: (0, 0)
    return pl.pallas_call(
        _head_body,
        grid=(_GRID,),
        in_specs=[
            pl.BlockSpec((_ROWS, 192), row),
            pl.BlockSpec((192, 512), full),
            pl.BlockSpec((1, 512), full),
            pl.BlockSpec((1, 512), full),
            pl.BlockSpec((1, 512), full),
            pl.BlockSpec((512, 250), full),
            pl.BlockSpec((1, 250), full),
        ],
        out_specs=pl.BlockSpec((_ROWS, 250), row),
        out_shape=jax.ShapeDtypeStruct((_N, 250), jnp.float32),
    )(xin, W1, b1, g, be, W2, b2)


def kernel(patch_embeddings, coordinates, edge_index, edge_attr,
           tp_W1, tp_b1, tp_g1, tp_be1, tp_W2, tp_b2, tp_g2, tp_be2,
           g1_Wq, g1_bq, g1_Wk, g1_bk, g1_Wv, g1_bv, g1_We, g1_Ws, g1_bs, g1_ng, g1_nb,
           g2_Wq, g2_bq, g2_Wk, g2_bk, g2_Wv, g2_bv, g2_We, g2_Ws, g2_bs, g2_ng, g2_nb,
           pr_W1, pr_b1, pr_g, pr_be, pr_W2, pr_b2):
    r1 = lambda a: a.reshape(1, -1)
    h, penc = _embed(patch_embeddings, coordinates,
                     tp_W1, r1(tp_b1), r1(tp_g1), r1(tp_be1),
                     tp_W2, r1(tp_b2), r1(tp_g2), r1(tp_be2))
    init = jnp.concatenate([h, penc], axis=1)
    src = edge_index[0]
    dst = edge_index[1]

    x = init
    for (Wq, bq, Wk, bk, Wv, bv, We, Ws, bs, ng, nb) in (
            (g1_Wq, g1_bq, g1_Wk, g1_bk, g1_Wv, g1_bv, g1_We, g1_Ws, g1_bs, g1_ng, g1_nb),
            (g2_Wq, g2_bq, g2_Wk, g2_bk, g2_Wv, g2_bv, g2_We, g2_Ws, g2_bs, g2_ng, g2_nb)):
        M = _mk_M(We)
        dtab, stab = _qkv_pack(x, Wq, r1(bq), Wk, r1(bk), Wv, r1(bv), M)
        parts = _edge_pass(dtab, stab, edge_attr, src, dst)
        SaccB = _SACC + jnp.zeros((_ACCW, _F), jnp.float32).at[100:116, :].set(M.T)
        x = _finalize(parts[0], parts[1], x, Ws, r1(bs), r1(ng), r1(nb), SaccB, _SDEN)

    ctx = x
    xin = jnp.concatenate([init, ctx], axis=1)
    out = _head(xin, pr_W1, r1(pr_b1), r1(pr_g), r1(pr_be), pr_W2, r1(pr_b2))
    return out, ctx


# speedup vs baseline: 29.3563x; 1.6083x over previous
import functools
import math

import jax
import jax.numpy as jnp
import numpy as np
from jax import lax
from jax.experimental import pallas as pl
from jax.experimental.pallas import tpu as pltpu
from jax.experimental.pallas import tpu_sc as plsc

_N = 10000
_E = 640000
_HEADS = 4
_HD = 24
_F = 96
_EDIM = 4

_ROWS = 1000
_GRID = _N // _ROWS

_NW = 32
_EPW = _E // _NW
_C = 16
_NCHUNK = _EPW // _C
_EPB = 2000
_CPB = _EPB // _C
_ACCW = 128
_ZCH = 16
_NZCH = _N // _ZCH


def _ln(x, g, b):
    m = jnp.mean(x, axis=-1, keepdims=True)
    v = jnp.mean((x - m) ** 2, axis=-1, keepdims=True)
    return g * (x - m) / jnp.sqrt(v + 1e-5) + b


_HEADCOL = np.repeat(np.eye(4, dtype=np.float32), 24, axis=0)
_SDEN = np.zeros((_ACCW, 96), np.float32)
_SDEN[112:116, :] = _HEADCOL.T
_SACC = np.zeros((_ACCW, 96), np.float32)
_SACC[:96, :] = np.eye(96, dtype=np.float32)

_OH = [np.eye(16, dtype=np.float32)[r] for r in range(16)]
_LO8 = np.concatenate([np.ones(8, np.float32), np.zeros(8, np.float32)])
_HI8 = 1.0 - _LO8
_G4 = [np.repeat(np.eye(4, dtype=np.float32)[h], 4) for h in range(4)]
_O4 = [np.eye(16, dtype=np.float32)[h] for h in range(4)]


def _mk_M(We):
    return (jnp.transpose(We).reshape(96, 1, 4) * _HEADCOL.reshape(96, 4, 1)).reshape(96, 16)


def _embed_body(pe, co, W1, b1, g1, be1, W2, b2, g2, be2, h_o, pe_o):
    x = pe[...]
    t = jnp.maximum(jnp.dot(x, W1[...]) + b1[...], 0.0)
    t = _ln(t, g1[...], be1[...])
    t = jnp.maximum(jnp.dot(t, W2[...]) + b2[...], 0.0)
    h_o[...] = _ln(t, g2[...], be2[...])
    c = co[...]
    lane = lax.broadcasted_iota(jnp.int32, (1, 32), 1)
    i16 = lane % 16
    freq = jnp.exp((i16 // 2).astype(jnp.float32) * (2.0 * -math.log(10000.0) / 16.0))
    phase = (i16 % 2).astype(jnp.float32) * (math.pi / 2.0)
    xsel = (lane < 16).astype(jnp.float32)
    c_sel = c[:, 0:1] * xsel + c[:, 1:2] * (1.0 - xsel)
    pe_o[...] = jnp.sin(c_sel * freq + phase)


def _embed(pe, co, W1, b1, g1, be1, W2, b2, g2, be2):
    row = lambda i: (i, 0)
    full = lambda i: (0, 0)
    return pl.pallas_call(
        _embed_body,
        grid=(_GRID,),
        in_specs=[
            pl.BlockSpec((_ROWS, 768), row),
            pl.BlockSpec((_ROWS, 2), row),
            pl.BlockSpec((768, 2048), full),
            pl.BlockSpec((1, 2048), full),
            pl.BlockSpec((1, 2048), full),
            pl.BlockSpec((1, 2048), full),
            pl.BlockSpec((2048, 64), full),
            pl.BlockSpec((1, 64), full),
            pl.BlockSpec((1, 64), full),
            pl.BlockSpec((1, 64), full),
        ],
        out_specs=[pl.BlockSpec((_ROWS, 64), row), pl.BlockSpec((_ROWS, 32), row)],
        out_shape=[
            jax.ShapeDtypeStruct((_N, 64), jnp.float32),
            jax.ShapeDtypeStruct((_N, 32), jnp.float32),
        ],
    )(pe, co, W1, b1, g1, be1, W2, b2, g2, be2)


def _qkv_body(x, Wq, bq, Wk, bk, Wv, bv, M, d_o, s_o):
    xv = x[...]
    q = jnp.dot(xv, Wq[...]) + bq[...]
    qe = jnp.dot(q, M[...])
    k = jnp.dot(xv, Wk[...]) + bk[...]
    v = jnp.dot(xv, Wv[...]) + bv[...]
    zp = jnp.zeros((xv.shape[0], 16), jnp.float32)
    d_o[...] = jnp.concatenate([q, qe, zp], axis=1)
    s_o[...] = jnp.concatenate([k, v, zp, zp, zp, zp], axis=1)


def _qkv_pack(x, Wq, bq, Wk, bk, Wv, bv, M):
    row = lambda i: (i, 0)
    full = lambda i: (0, 0)
    return pl.pallas_call(
        _qkv_body,
        grid=(_GRID,),
        in_specs=[
            pl.BlockSpec((_ROWS, _F), row),
            pl.BlockSpec((_F, _F), full),
            pl.BlockSpec((1, _F), full),
            pl.BlockSpec((_F, _F), full),
            pl.BlockSpec((1, _F), full),
            pl.BlockSpec((_F, _F), full),
            pl.BlockSpec((1, _F), full),
            pl.BlockSpec((_F, 16), full),
        ],
        out_specs=[
            pl.BlockSpec((_ROWS, 128), row),
            pl.BlockSpec((_ROWS, 256), row),
        ],
        out_shape=[
            jax.ShapeDtypeStruct((_N, 128), jnp.float32),
            jax.ShapeDtypeStruct((_N, 256), jnp.float32),
        ],
    )(x, Wq, bq, Wk, bk, Wv, bv, M)


def _edge_body(dtab, stab, eaf, src, dst, out,
               sidx, didx, eab, drow, srow, scaled, acc,
               semd0, semd1, sems0, sems1, sema0, sema1):
    cid = lax.axis_index("c")
    sid = lax.axis_index("s")
    wid = sid * 2 + cid

    z16 = jnp.zeros((16,), jnp.float32)
    lanes = lax.iota(jnp.int32, 16)
    inv = jnp.float32(1.0 / math.sqrt(_HD))

    def zrow(r, _):
        for j in range(_ACCW // 16):
            scaled[r, pl.ds(j * 16, 16)] = z16
        return 0

    lax.fori_loop(0, 2 * _C, zrow, 0)

    def zchunk(j, _):
        idx = sid + 16 * j

        @pl.when(idx < _NZCH)
        def _():
            off = pl.multiple_of(idx * _ZCH, 8)
            pltpu.sync_copy(scaled.at[pl.ds(0, _ZCH)], acc.at[pl.ds(off, _ZCH)])

        return 0

    lax.fori_loop(0, (_NZCH + 15) // 16, zchunk, 0)
    plsc.subcore_barrier()

    zi16 = jnp.zeros((16,), jnp.int32)

    def step(j, carry):
        dprev, ea0, ea1, ea2, ea3 = carry
        par = j % 2
        parc = (j - 1) % 2
        blk = j // _CPB
        jbe = j - blk * _CPB

        @pl.when(jnp.logical_and(j < _NCHUNK, jbe == 0))
        def _():
            base = pl.multiple_of(wid * _EPW + blk * _EPB, 8)
            pltpu.sync_copy(src.at[pl.ds(base, _EPB)], sidx)
            pltpu.sync_copy(dst.at[pl.ds(base, _EPB)], didx)
            pltpu.sync_copy(eaf.at[pl.ds(base * _EDIM, _EPB * _EDIM)], eab)

        inb = j < _NCHUNK
        jsafe = jnp.where(inb, jbe, 0)
        cbase = jsafe * _C + lanes
        svec = plsc.load_gather(sidx, [cbase])
        dvec = plsc.load_gather(didx, [cbase])
        eidx = cbase * _EDIM
        eacn = [plsc.load_gather(eab, [eidx + d]) for d in range(_EDIM)]

        @pl.when(jnp.logical_and(j >= 1, parc == 0))
        def _():
            pltpu.make_async_copy(dtab.at[dprev], drow.at[pl.ds(0, _C)],
                                  semd0).wait()
            pltpu.make_async_copy(stab.at[dprev], srow.at[pl.ds(0, _C)],
                                  sems0).wait()

        @pl.when(jnp.logical_and(j >= 1, parc == 1))
        def _():
            pltpu.make_async_copy(dtab.at[dprev], drow.at[pl.ds(_C, _C)],
                                  semd1).wait()
            pltpu.make_async_copy(stab.at[dprev], srow.at[pl.ds(_C, _C)],
                                  sems1).wait()

        @pl.when(jnp.logical_and(inb, par == 0))
        def _():
            pltpu.async_copy(dtab.at[dvec], drow.at[pl.ds(0, _C)], semd0)
            pltpu.async_copy(stab.at[svec], srow.at[pl.ds(0, _C)], sems0)

        @pl.when(jnp.logical_and(inb, par == 1))
        def _():
            pltpu.async_copy(dtab.at[dvec], drow.at[pl.ds(_C, _C)], semd1)
            pltpu.async_copy(stab.at[svec], srow.at[pl.ds(_C, _C)], sems1)

        @pl.when(j >= 1)
        def _():
            rl = lanes + parc * _C

            @pl.when(jnp.logical_and(j >= 3, parc == 0))
            def _():
                pltpu.make_async_copy(scaled.at[pl.ds(0, _C)],
                                      acc.at[dprev], sema0).wait()

            @pl.when(jnp.logical_and(j >= 3, parc == 1))
            def _():
                pltpu.make_async_copy(scaled.at[pl.ds(_C, _C)],
                                      acc.at[dprev], sema1).wait()

            a = [z16, z16, z16, z16]
            for c in range(_F):
                colv = jnp.full((16,), c, jnp.int32)
                qc = plsc.load_gather(drow, [rl, colv])
                kc = plsc.load_gather(srow, [rl, colv])
                a[c // _HD] = a[c // _HD] + qc * kc
            eac = [ea0, ea1, ea2, ea3]
            ex = []
            for h in range(_HEADS):
                ah = a[h]
                for d in range(_EDIM):
                    qe = plsc.load_gather(
                        drow, [rl, jnp.full((16,), _F + h * 4 + d, jnp.int32)])
                    ah = ah + eac[d] * qe
                ex.append(jnp.exp(ah * inv))
            base_row = parc * _C
            lo8 = (lanes < 8).astype(jnp.float32)
            hi8 = 1.0 - lo8
            g4 = [(lanes // 4 == h).astype(jnp.float32) for h in range(_HEADS)]
            o4 = [(lanes == h).astype(jnp.float32) for h in range(_HEADS)]
            m4 = [(lanes % 4 == d).astype(jnp.float32) for d in range(_EDIM)]
            for r in range(_C):
                row = base_row + r
                oh = (lanes == r).astype(jnp.float32)
                exs = [jnp.sum(ex[h] * oh) for h in range(_HEADS)]
                eass = [jnp.sum(eac[d] * oh) for d in range(_EDIM)]
                E = [jnp.full((16,), exs[h]) for h in range(_HEADS)]
                exm = [E[0], E[0] * lo8 + E[1] * hi8, E[1],
                       E[2], E[2] * lo8 + E[3] * hi8, E[3]]
                for jj in range(6):
                    scaled[row, pl.ds(jj * 16, 16)] = (
                        srow[row, pl.ds(_F + jj * 16, 16)] * exm[jj])
                eav = sum(jnp.full((16,), eass[d]) * m4[d] for d in range(_EDIM))
                exh4 = sum(E[h] * g4[h] for h in range(_HEADS))
                scaled[row, pl.ds(96, 16)] = exh4 * eav
                scaled[row, pl.ds(112, 16)] = sum(
                    E[h] * o4[h] for h in range(_HEADS))
            @pl.when(parc == 0)
            def _():
                pltpu.async_copy(scaled.at[pl.ds(0, _C)], acc.at[dprev],
                                 sema0, add=True)

            @pl.when(parc == 1)
            def _():
                pltpu.async_copy(scaled.at[pl.ds(_C, _C)], acc.at[dprev],
                                 sema1, add=True)

        return (dvec, eacn[0], eacn[1], eacn[2], eacn[3])

    lax.fori_loop(0, _NCHUNK + 1, step,
                  (zi16, z16, z16, z16, z16))
    pltpu.make_async_copy(scaled.at[pl.ds(0, _C)], acc.at[lanes], sema0).wait()
    pltpu.make_async_copy(scaled.at[pl.ds(_C, _C)], acc.at[lanes], sema1).wait()
    plsc.subcore_barrier()

    def fchunk(j, _):
        idx = sid + 16 * j

        @pl.when(idx < _NZCH)
        def _():
            off = pl.multiple_of(idx * _ZCH, 8)
            pltpu.sync_copy(acc.at[pl.ds(off, _ZCH)],
                            out.at[cid, pl.ds(off, _ZCH)])

        return 0

    lax.fori_loop(0, (_NZCH + 15) // 16, fchunk, 0)


def _edge_pass(dtab, stab, ea, src, dst):
    mesh = plsc.VectorSubcoreMesh(core_axis_name="c", subcore_axis_name="s")
    kern = pl.kernel(
        _edge_body,
        out_type=jax.ShapeDtypeStruct((2, _N, _ACCW), jnp.float32),
        mesh=mesh,
        compiler_params=pltpu.CompilerParams(needs_layout_passes=False),
        scratch_types=[
            pltpu.VMEM((_EPB,), jnp.int32),
            pltpu.VMEM((_EPB,), jnp.int32),
            pltpu.VMEM((_EPB * _EDIM,), jnp.float32),
            pltpu.VMEM((2 * _C, 128), jnp.float32),
            pltpu.VMEM((2 * _C, 256), jnp.float32),
            pltpu.VMEM((2 * _C, _ACCW), jnp.float32),
            pltpu.VMEM_SHARED((_N, _ACCW), jnp.float32),
            pltpu.SemaphoreType.DMA,
            pltpu.SemaphoreType.DMA,
            pltpu.SemaphoreType.DMA,
            pltpu.SemaphoreType.DMA,
            pltpu.SemaphoreType.DMA,
            pltpu.SemaphoreType.DMA,
        ],
    )
    return kern(dtab, stab, ea.reshape(-1), src, dst)


def _fin_body(p0, p1, x, Ws, bs, ng, nb, SaccB, Sden, o):
    s = p0[...] + p1[...]
    num = jnp.dot(s, SaccB[...])
    den = jnp.dot(s, Sden[...])
    agg = num / (den + 1e-16)
    xv = x[...]
    y = agg + jnp.dot(xv, Ws[...]) + bs[...]
    y = jnp.maximum(y, 0.0)
    y = _ln(y, ng[...], nb[...])
    o[...] = xv + y


def _finalize(p0, p1, x, Ws, bs, ng, nb, SaccB, Sden):
    row = lambda i: (i, 0)
    full = lambda i: (0, 0)
    return pl.pallas_call(
        _fin_body,
        grid=(_GRID,),
        in_specs=[
            pl.BlockSpec((_ROWS, _ACCW), row),
            pl.BlockSpec((_ROWS, _ACCW), row),
            pl.BlockSpec((_ROWS, _F), row),
            pl.BlockSpec((_F, _F), full),
            pl.BlockSpec((1, _F), full),
            pl.BlockSpec((1, _F), full),
            pl.BlockSpec((1, _F), full),
            pl.BlockSpec((_ACCW, _F), full),
            pl.BlockSpec((_ACCW, _F), full),
        ],
        out_specs=pl.BlockSpec((_ROWS, _F), row),
        out_shape=jax.ShapeDtypeStruct((_N, _F), jnp.float32),
    )(p0, p1, x, Ws, bs, ng, nb, SaccB, Sden)


def _head_body(xin, W1, b1, g, be, W2, b2, o):
    y = jnp.maximum(jnp.dot(xin[...], W1[...]) + b1[...], 0.0)
    y = _ln(y, g[...], be[...])
    o[...] = jnp.dot(y, W2[...]) + b2[...]


def _head(xin, W1, b1, g, be, W2, b2):
    row = lambda i: (i, 0)
    full = lambda i: (0, 0)
    return pl.pallas_call(
        _head_body,
        grid=(_GRID,),
        in_specs=[
            pl.BlockSpec((_ROWS, 192), row),
            pl.BlockSpec((192, 512), full),
            pl.BlockSpec((1, 512), full),
            pl.BlockSpec((1, 512), full),
            pl.BlockSpec((1, 512), full),
            pl.BlockSpec((512, 250), full),
            pl.BlockSpec((1, 250), full),
        ],
        out_specs=pl.BlockSpec((_ROWS, 250), row),
        out_shape=jax.ShapeDtypeStruct((_N, 250), jnp.float32),
    )(xin, W1, b1, g, be, W2, b2)


def kernel(patch_embeddings, coordinates, edge_index, edge_attr,
           tp_W1, tp_b1, tp_g1, tp_be1, tp_W2, tp_b2, tp_g2, tp_be2,
           g1_Wq, g1_bq, g1_Wk, g1_bk, g1_Wv, g1_bv, g1_We, g1_Ws, g1_bs, g1_ng, g1_nb,
           g2_Wq, g2_bq, g2_Wk, g2_bk, g2_Wv, g2_bv, g2_We, g2_Ws, g2_bs, g2_ng, g2_nb,
           pr_W1, pr_b1, pr_g, pr_be, pr_W2, pr_b2):
    r1 = lambda a: a.reshape(1, -1)
    h, penc = _embed(patch_embeddings, coordinates,
                     tp_W1, r1(tp_b1), r1(tp_g1), r1(tp_be1),
                     tp_W2, r1(tp_b2), r1(tp_g2), r1(tp_be2))
    init = jnp.concatenate([h, penc], axis=1)
    src = edge_index[0]
    dst = edge_index[1]

    x = init
    for (Wq, bq, Wk, bk, Wv, bv, We, Ws, bs, ng, nb) in (
            (g1_Wq, g1_bq, g1_Wk, g1_bk, g1_Wv, g1_bv, g1_We, g1_Ws, g1_bs, g1_ng, g1_nb),
            (g2_Wq, g2_bq, g2_Wk, g2_bk, g2_Wv, g2_bv, g2_We, g2_Ws, g2_bs, g2_ng, g2_nb)):
        M = _mk_M(We)
        dtab, stab = _qkv_pack(x, Wq, r1(bq), Wk, r1(bk), Wv, r1(bv), M)
        parts = _edge_pass(dtab, stab, edge_attr, src, dst)
        SaccB = _SACC + jnp.zeros((_ACCW, _F), jnp.float32).at[96:112, :].set(M.T)
        x = _finalize(parts[0], parts[1], x, Ws, r1(bs), r1(ng), r1(nb), SaccB, _SDEN)

    ctx = x
    xin = jnp.concatenate([init, ctx], axis=1)
    out = _head(xin, pr_W1, r1(pr_b1), r1(pr_g), r1(pr_be), pr_W2, r1(pr_b2))
    return out, ctx


# speedup vs baseline: 29.4251x; 1.0023x over previous
import functools
import math

import jax
import jax.numpy as jnp
import numpy as np
from jax import lax
from jax.experimental import pallas as pl
from jax.experimental.pallas import tpu as pltpu
from jax.experimental.pallas import tpu_sc as plsc

_N = 10000
_E = 640000
_HEADS = 4
_HD = 24
_F = 96
_EDIM = 4

_ROWS = 1000
_GRID = _N // _ROWS

_NW = 32
_EPW = _E // _NW
_C = 16
_NCHUNK = _EPW // _C
_EPB = 2000
_CPB = _EPB // _C
_ACCW = 128
_ZCH = 16
_NZCH = _N // _ZCH


def _ln(x, g, b):
    m = jnp.mean(x, axis=-1, keepdims=True)
    v = jnp.mean((x - m) ** 2, axis=-1, keepdims=True)
    return g * (x - m) / jnp.sqrt(v + 1e-5) + b


_HEADCOL = np.repeat(np.eye(4, dtype=np.float32), 24, axis=0)
_SDEN = np.zeros((_ACCW, 96), np.float32)
_SDEN[112:116, :] = _HEADCOL.T
_SACC = np.zeros((_ACCW, 96), np.float32)
_SACC[:96, :] = np.eye(96, dtype=np.float32)

_OH = [np.eye(16, dtype=np.float32)[r] for r in range(16)]
_LO8 = np.concatenate([np.ones(8, np.float32), np.zeros(8, np.float32)])
_HI8 = 1.0 - _LO8
_G4 = [np.repeat(np.eye(4, dtype=np.float32)[h], 4) for h in range(4)]
_O4 = [np.eye(16, dtype=np.float32)[h] for h in range(4)]


def _mk_M(We):
    return (jnp.transpose(We).reshape(96, 1, 4) * _HEADCOL.reshape(96, 4, 1)).reshape(96, 16)


def _embed_body(pe, co, W1, b1, g1, be1, W2, b2, g2, be2, h_o, pe_o):
    x = pe[...]
    t = jnp.maximum(jnp.dot(x, W1[...]) + b1[...], 0.0)
    t = _ln(t, g1[...], be1[...])
    t = jnp.maximum(jnp.dot(t, W2[...]) + b2[...], 0.0)
    h_o[...] = _ln(t, g2[...], be2[...])
    c = co[...]
    lane = lax.broadcasted_iota(jnp.int32, (1, 32), 1)
    i16 = lane % 16
    freq = jnp.exp((i16 // 2).astype(jnp.float32) * (2.0 * -math.log(10000.0) / 16.0))
    phase = (i16 % 2).astype(jnp.float32) * (math.pi / 2.0)
    xsel = (lane < 16).astype(jnp.float32)
    c_sel = c[:, 0:1] * xsel + c[:, 1:2] * (1.0 - xsel)
    pe_o[...] = jnp.sin(c_sel * freq + phase)


def _embed(pe, co, W1, b1, g1, be1, W2, b2, g2, be2):
    row = lambda i: (i, 0)
    full = lambda i: (0, 0)
    return pl.pallas_call(
        _embed_body,
        grid=(_GRID,),
        in_specs=[
            pl.BlockSpec((_ROWS, 768), row),
            pl.BlockSpec((_ROWS, 2), row),
            pl.BlockSpec((768, 2048), full),
            pl.BlockSpec((1, 2048), full),
            pl.BlockSpec((1, 2048), full),
            pl.BlockSpec((1, 2048), full),
            pl.BlockSpec((2048, 64), full),
            pl.BlockSpec((1, 64), full),
            pl.BlockSpec((1, 64), full),
            pl.BlockSpec((1, 64), full),
        ],
        out_specs=[pl.BlockSpec((_ROWS, 64), row), pl.BlockSpec((_ROWS, 32), row)],
        out_shape=[
            jax.ShapeDtypeStruct((_N, 64), jnp.float32),
            jax.ShapeDtypeStruct((_N, 32), jnp.float32),
        ],
    )(pe, co, W1, b1, g1, be1, W2, b2, g2, be2)


def _qkv_body(x, Wq, bq, Wk, bk, Wv, bv, M, d_o, s_o):
    xv = x[...]
    q = jnp.dot(xv, Wq[...]) + bq[...]
    qe = jnp.dot(q, M[...])
    k = jnp.dot(xv, Wk[...]) + bk[...]
    v = jnp.dot(xv, Wv[...]) + bv[...]
    zp = jnp.zeros((xv.shape[0], 16), jnp.float32)
    d_o[...] = jnp.concatenate([q, qe, zp], axis=1)
    s_o[...] = jnp.concatenate([k, v, zp, zp, zp, zp], axis=1)


def _qkv_pack(x, Wq, bq, Wk, bk, Wv, bv, M):
    row = lambda i: (i, 0)
    full = lambda i: (0, 0)
    return pl.pallas_call(
        _qkv_body,
        grid=(_GRID,),
        in_specs=[
            pl.BlockSpec((_ROWS, _F), row),
            pl.BlockSpec((_F, _F), full),
            pl.BlockSpec((1, _F), full),
            pl.BlockSpec((_F, _F), full),
            pl.BlockSpec((1, _F), full),
            pl.BlockSpec((_F, _F), full),
            pl.BlockSpec((1, _F), full),
            pl.BlockSpec((_F, 16), full),
        ],
        out_specs=[
            pl.BlockSpec((_ROWS, 128), row),
            pl.BlockSpec((_ROWS, 256), row),
        ],
        out_shape=[
            jax.ShapeDtypeStruct((_N, 128), jnp.float32),
            jax.ShapeDtypeStruct((_N, 256), jnp.float32),
        ],
    )(x, Wq, bq, Wk, bk, Wv, bv, M)


def _edge_body(dtab, stab, eaf, src, dst, out,
               sidx, didx, eab, drow, srow, scaled, acc,
               semd0, semd1, sems0, sems1, sema0, sema1):
    cid = lax.axis_index("c")
    sid = lax.axis_index("s")
    wid = sid * 2 + cid

    z16 = jnp.zeros((16,), jnp.float32)
    lanes = lax.iota(jnp.int32, 16)
    inv = jnp.float32(1.0 / math.sqrt(_HD))

    def zrow(r, _):
        for j in range(_ACCW // 16):
            scaled[r, pl.ds(j * 16, 16)] = z16
        return 0

    lax.fori_loop(0, 2 * _C, zrow, 0)

    def zchunk(j, _):
        idx = sid + 16 * j

        @pl.when(idx < _NZCH)
        def _():
            off = pl.multiple_of(idx * _ZCH, 8)
            pltpu.sync_copy(scaled.at[pl.ds(0, _ZCH)], acc.at[pl.ds(off, _ZCH)])

        return 0

    lax.fori_loop(0, (_NZCH + 15) // 16, zchunk, 0)
    plsc.subcore_barrier()

    zi16 = jnp.zeros((16,), jnp.int32)

    def step(j, carry):
        dprev, ea0, ea1, ea2, ea3 = carry
        par = j % 2
        parc = (j - 1) % 2
        blk = j // _CPB
        jbe = j - blk * _CPB

        @pl.when(jnp.logical_and(j < _NCHUNK, jbe == 0))
        def _():
            base = pl.multiple_of(wid * _EPW + blk * _EPB, 8)
            pltpu.sync_copy(src.at[pl.ds(base, _EPB)], sidx)
            pltpu.sync_copy(dst.at[pl.ds(base, _EPB)], didx)
            pltpu.sync_copy(eaf.at[pl.ds(base * _EDIM, _EPB * _EDIM)], eab)

        inb = j < _NCHUNK
        jsafe = jnp.where(inb, jbe, 0)
        cbase = jsafe * _C + lanes
        svec = plsc.load_gather(sidx, [cbase])
        dvec = plsc.load_gather(didx, [cbase])
        eidx = cbase * _EDIM
        eacn = [plsc.load_gather(eab, [eidx + d]) for d in range(_EDIM)]

        @pl.when(jnp.logical_and(j >= 1, parc == 0))
        def _():
            pltpu.make_async_copy(dtab.at[dprev], drow.at[pl.ds(0, _C)],
                                  semd0).wait()
            pltpu.make_async_copy(stab.at[dprev], srow.at[pl.ds(0, _C)],
                                  sems0).wait()

        @pl.when(jnp.logical_and(j >= 1, parc == 1))
        def _():
            pltpu.make_async_copy(dtab.at[dprev], drow.at[pl.ds(_C, _C)],
                                  semd1).wait()
            pltpu.make_async_copy(stab.at[dprev], srow.at[pl.ds(_C, _C)],
                                  sems1).wait()

        @pl.when(jnp.logical_and(inb, par == 0))
        def _():
            pltpu.async_copy(dtab.at[dvec], drow.at[pl.ds(0, _C)], semd0)
            pltpu.async_copy(stab.at[svec], srow.at[pl.ds(0, _C)], sems0)

        @pl.when(jnp.logical_and(inb, par == 1))
        def _():
            pltpu.async_copy(dtab.at[dvec], drow.at[pl.ds(_C, _C)], semd1)
            pltpu.async_copy(stab.at[svec], srow.at[pl.ds(_C, _C)], sems1)

        @pl.when(j >= 1)
        def _():
            rl = lanes + parc * _C

            @pl.when(jnp.logical_and(j >= 3, parc == 0))
            def _():
                pltpu.make_async_copy(scaled.at[pl.ds(0, _C)],
                                      acc.at[dprev], sema0).wait()

            @pl.when(jnp.logical_and(j >= 3, parc == 1))
            def _():
                pltpu.make_async_copy(scaled.at[pl.ds(_C, _C)],
                                      acc.at[dprev], sema1).wait()

            eac = [ea0, ea1, ea2, ea3]
            base_row = parc * _C
            lo8 = (lanes < 8).astype(jnp.float32)
            hi8 = 1.0 - lo8
            g4 = [(lanes // 4 == h).astype(jnp.float32) for h in range(_HEADS)]
            o4 = [(lanes == h).astype(jnp.float32) for h in range(_HEADS)]
            m4 = [(lanes % 4 == d).astype(jnp.float32) for d in range(_EDIM)]
            for r in range(_C):
                row = base_row + r
                oh = (lanes == r).astype(jnp.float32)
                eass = [jnp.sum(eac[d] * oh) for d in range(_EDIM)]
                eav = sum(jnp.full((16,), eass[d]) * m4[d] for d in range(_EDIM))
                p = [drow[row, pl.ds(jj * 16, 16)] * srow[row, pl.ds(jj * 16, 16)]
                     for jj in range(6)]
                t0 = jnp.sum(p[0])
                t1 = jnp.sum(p[1])
                t1l = jnp.sum(p[1] * lo8)
                t2 = jnp.sum(p[2])
                t3 = jnp.sum(p[3])
                t4 = jnp.sum(p[4])
                t4l = jnp.sum(p[4] * lo8)
                t5 = jnp.sum(p[5])
                te = drow[row, pl.ds(_F, 16)] * eav
                s = [t0 + t1l, t1 - t1l + t2, t3 + t4l, t4 - t4l + t5]
                E = [jnp.exp(jnp.full((16,), (s[h] + jnp.sum(te * g4[h])) * inv))
                     for h in range(_HEADS)]
                exm = [E[0], E[0] * lo8 + E[1] * hi8, E[1],
                       E[2], E[2] * lo8 + E[3] * hi8, E[3]]
                for jj in range(6):
                    scaled[row, pl.ds(jj * 16, 16)] = (
                        srow[row, pl.ds(_F + jj * 16, 16)] * exm[jj])
                exh4 = sum(E[h] * g4[h] for h in range(_HEADS))
                scaled[row, pl.ds(96, 16)] = exh4 * eav
                scaled[row, pl.ds(112, 16)] = sum(
                    E[h] * o4[h] for h in range(_HEADS))
            @pl.when(parc == 0)
            def _():
                pltpu.async_copy(scaled.at[pl.ds(0, _C)], acc.at[dprev],
                                 sema0, add=True)

            @pl.when(parc == 1)
            def _():
                pltpu.async_copy(scaled.at[pl.ds(_C, _C)], acc.at[dprev],
                                 sema1, add=True)

        return (dvec, eacn[0], eacn[1], eacn[2], eacn[3])

    lax.fori_loop(0, _NCHUNK + 1, step,
                  (zi16, z16, z16, z16, z16))
    pltpu.make_async_copy(scaled.at[pl.ds(0, _C)], acc.at[lanes], sema0).wait()
    pltpu.make_async_copy(scaled.at[pl.ds(_C, _C)], acc.at[lanes], sema1).wait()
    plsc.subcore_barrier()

    def fchunk(j, _):
        idx = sid + 16 * j

        @pl.when(idx < _NZCH)
        def _():
            off = pl.multiple_of(idx * _ZCH, 8)
            pltpu.sync_copy(acc.at[pl.ds(off, _ZCH)],
                            out.at[cid, pl.ds(off, _ZCH)])

        return 0

    lax.fori_loop(0, (_NZCH + 15) // 16, fchunk, 0)


def _edge_pass(dtab, stab, ea, src, dst):
    mesh = plsc.VectorSubcoreMesh(core_axis_name="c", subcore_axis_name="s")
    kern = pl.kernel(
        _edge_body,
        out_type=jax.ShapeDtypeStruct((2, _N, _ACCW), jnp.float32),
        mesh=mesh,
        compiler_params=pltpu.CompilerParams(needs_layout_passes=False),
        scratch_types=[
            pltpu.VMEM((_EPB,), jnp.int32),
            pltpu.VMEM((_EPB,), jnp.int32),
            pltpu.VMEM((_EPB * _EDIM,), jnp.float32),
            pltpu.VMEM((2 * _C, 128), jnp.float32),
            pltpu.VMEM((2 * _C, 256), jnp.float32),
            pltpu.VMEM((2 * _C, _ACCW), jnp.float32),
            pltpu.VMEM_SHARED((_N, _ACCW), jnp.float32),
            pltpu.SemaphoreType.DMA,
            pltpu.SemaphoreType.DMA,
            pltpu.SemaphoreType.DMA,
            pltpu.SemaphoreType.DMA,
            pltpu.SemaphoreType.DMA,
            pltpu.SemaphoreType.DMA,
        ],
    )
    return kern(dtab, stab, ea.reshape(-1), src, dst)


def _fin_body(p0, p1, x, Ws, bs, ng, nb, SaccB, Sden, o):
    s = p0[...] + p1[...]
    num = jnp.dot(s, SaccB[...])
    den = jnp.dot(s, Sden[...])
    agg = num / (den + 1e-16)
    xv = x[...]
    y = agg + jnp.dot(xv, Ws[...]) + bs[...]
    y = jnp.maximum(y, 0.0)
    y = _ln(y, ng[...], nb[...])
    o[...] = xv + y


def _finalize(p0, p1, x, Ws, bs, ng, nb, SaccB, Sden):
    row = lambda i: (i, 0)
    full = lambda i: (0, 0)
    return pl.pallas_call(
        _fin_body,
        grid=(_GRID,),
        in_specs=[
            pl.BlockSpec((_ROWS, _ACCW), row),
            pl.BlockSpec((_ROWS, _ACCW), row),
            pl.BlockSpec((_ROWS, _F), row),
            pl.BlockSpec((_F, _F), full),
            pl.BlockSpec((1, _F), full),
            pl.BlockSpec((1, _F), full),
            pl.BlockSpec((1, _F), full),
            pl.BlockSpec((_ACCW, _F), full),
            pl.BlockSpec((_ACCW, _F), full),
        ],
        out_specs=pl.BlockSpec((_ROWS, _F), row),
        out_shape=jax.ShapeDtypeStruct((_N, _F), jnp.float32),
    )(p0, p1, x, Ws, bs, ng, nb, SaccB, Sden)


def _head_body(xin, W1, b1, g, be, W2, b2, o):
    y = jnp.maximum(jnp.dot(xin[...], W1[...]) + b1[...], 0.0)
    y = _ln(y, g[...], be[...])
    o[...] = jnp.dot(y, W2[...]) + b2[...]


def _head(xin, W1, b1, g, be, W2, b2):
    row = lambda i: (i, 0)
    full = lambda i: (0, 0)
    return pl.pallas_call(
        _head_body,
        grid=(_GRID,),
        in_specs=[
            pl.BlockSpec((_ROWS, 192), row),
            pl.BlockSpec((192, 512), full),
            pl.BlockSpec((1, 512), full),
            pl.BlockSpec((1, 512), full),
            pl.BlockSpec((1, 512), full),
            pl.BlockSpec((512, 250), full),
            pl.BlockSpec((1, 250), full),
        ],
        out_specs=pl.BlockSpec((_ROWS, 250), row),
        out_shape=jax.ShapeDtypeStruct((_N, 250), jnp.float32),
    )(xin, W1, b1, g, be, W2, b2)


def kernel(patch_embeddings, coordinates, edge_index, edge_attr,
           tp_W1, tp_b1, tp_g1, tp_be1, tp_W2, tp_b2, tp_g2, tp_be2,
           g1_Wq, g1_bq, g1_Wk, g1_bk, g1_Wv, g1_bv, g1_We, g1_Ws, g1_bs, g1_ng, g1_nb,
           g2_Wq, g2_bq, g2_Wk, g2_bk, g2_Wv, g2_bv, g2_We, g2_Ws, g2_bs, g2_ng, g2_nb,
           pr_W1, pr_b1, pr_g, pr_be, pr_W2, pr_b2):
    r1 = lambda a: a.reshape(1, -1)
    h, penc = _embed(patch_embeddings, coordinates,
                     tp_W1, r1(tp_b1), r1(tp_g1), r1(tp_be1),
                     tp_W2, r1(tp_b2), r1(tp_g2), r1(tp_be2))
    init = jnp.concatenate([h, penc], axis=1)
    src = edge_index[0]
    dst = edge_index[1]

    x = init
    for (Wq, bq, Wk, bk, Wv, bv, We, Ws, bs, ng, nb) in (
            (g1_Wq, g1_bq, g1_Wk, g1_bk, g1_Wv, g1_bv, g1_We, g1_Ws, g1_bs, g1_ng, g1_nb),
            (g2_Wq, g2_bq, g2_Wk, g2_bk, g2_Wv, g2_bv, g2_We, g2_Ws, g2_bs, g2_ng, g2_nb)):
        M = _mk_M(We)
        dtab, stab = _qkv_pack(x, Wq, r1(bq), Wk, r1(bk), Wv, r1(bv), M)
        parts = _edge_pass(dtab, stab, edge_attr, src, dst)
        SaccB = _SACC + jnp.zeros((_ACCW, _F), jnp.float32).at[96:112, :].set(M.T)
        x = _finalize(parts[0], parts[1], x, Ws, r1(bs), r1(ng), r1(nb), SaccB, _SDEN)

    ctx = x
    xin = jnp.concatenate([init, ctx], axis=1)
    out = _head(xin, pr_W1, r1(pr_b1), r1(pr_g), r1(pr_be), pr_W2, r1(pr_b2))
    return out, ctx
